# baseline (speedup 1.0000x reference)
import functools

import jax
import jax.numpy as jnp
from jax import lax
from jax.experimental import pallas as pl
from jax.experimental.pallas import tpu as pltpu
from jax.experimental.pallas import tpu_sc as plsc

_PAD = 1
_NC, _NS = 2, 16
_NW = _NC * _NS
_D = 64
_STREAM = 128
_CHUNK_STREAMS = 8
_CHUNK = _STREAM * _CHUNK_STREAMS


def _make_gather(B, V):
    per_w = B // _NW
    n_chunks = per_w // _CHUNK
    assert per_w % _CHUNK == 0

    mesh = plsc.VectorSubcoreMesh(core_axis_name="c", subcore_axis_name="s")

    @functools.partial(
        pl.kernel,
        out_type=jax.ShapeDtypeStruct((B, _D), jnp.float32),
        mesh=mesh,
        scratch_types=[
            pltpu.VMEM((_CHUNK_STREAMS, _STREAM), jnp.int32),
            pltpu.VMEM((_CHUNK, _D), jnp.float32),
            pltpu.SemaphoreType.DMA,
        ],
        compiler_params=pltpu.CompilerParams(use_tc_tiling_on_sc=False, needs_layout_passes=False),
    )
    def gather_kernel(weight_hbm, idx_hbm, out_hbm, idx_v, rows_v, sem):
        wid = lax.axis_index("s") * _NC + lax.axis_index("c")
        lane = lax.iota(jnp.int32, 16)
        zero16 = jnp.zeros((16,), jnp.float32)

        def chunk_body(t, carry):
            base8 = wid * (per_w // _STREAM) + t * _CHUNK_STREAMS
            out_base = wid * per_w + t * _CHUNK
            pltpu.sync_copy(idx_hbm.at[pl.ds(base8, _CHUNK_STREAMS)], idx_v)
            copies = [
                pltpu.async_copy(
                    weight_hbm.at[idx_v.at[j]],
                    rows_v.at[pl.ds(j * _STREAM, _STREAM)],
                    sem,
                )
                for j in range(_CHUNK_STREAMS)
            ]
            m_any = idx_v[0, pl.ds(0, 16)] == _PAD
            for g in range(1, _CHUNK // 16):
                r, o = divmod(g * 16, _STREAM)
                m_any = m_any | (idx_v[r, pl.ds(o, 16)] == _PAD)
            n_pad = jnp.sum(m_any.astype(jnp.int32))
            for cp in copies:
                cp.wait()

            @pl.when(n_pad > 0)
            def _fix():
                for r in range(_CHUNK_STREAMS):
                    def fix_group(g, c):
                        idx16 = idx_v[r, pl.ds(g * 16, 16)]
                        m = idx16 == _PAD
                        @pl.when(jnp.sum(m.astype(jnp.int32)) > 0)
                        def _zero_rows():
                            rows16 = r * _STREAM + g * 16 + lane
                            for col in range(_D):
                                plsc.store_scatter(
                                    rows_v,
                                    [rows16, jnp.full((16,), col, jnp.int32)],
                                    zero16,
                                    mask=m,
                                )
                        return c
                    lax.fori_loop(0, _STREAM // 16, fix_group, 0)

            pltpu.sync_copy(rows_v, out_hbm.at[pl.ds(out_base, _CHUNK)])
            return carry

        lax.fori_loop(0, n_chunks, chunk_body, 0)

    return gather_kernel


def kernel(indices, weight):
    B = indices.size
    V, D = weight.shape
    assert D == _D
    idx2d = indices.reshape(B // _STREAM, _STREAM).astype(jnp.int32)
    out = _make_gather(B, V)(weight, idx2d)
    return out.reshape(indices.shape + (D,))


# speedup vs baseline: 1.0180x; 1.0180x over previous
import functools

import jax
import jax.numpy as jnp
from jax import lax
from jax.experimental import pallas as pl
from jax.experimental.pallas import tpu as pltpu
from jax.experimental.pallas import tpu_sc as plsc

_PAD = 1
_NC, _NS = 2, 16
_NW = _NC * _NS
_D = 64
_STREAM = 128
_CHUNK_STREAMS = 2
_CHUNK = _STREAM * _CHUNK_STREAMS
_NBUF = 4
_PF = 2


def _make_gather(B, V):
    per_w = B // _NW
    n_chunks = per_w // _CHUNK
    idx_rows = per_w // _STREAM
    assert per_w % _CHUNK == 0 and n_chunks % _NBUF == 0

    mesh = plsc.VectorSubcoreMesh(core_axis_name="c", subcore_axis_name="s")

    @functools.partial(
        pl.kernel,
        out_type=jax.ShapeDtypeStruct((B, _D), jnp.float32),
        mesh=mesh,
        scratch_types=[
            pltpu.VMEM((idx_rows, _STREAM), jnp.int32),
            pltpu.VMEM((_NBUF, _CHUNK, _D), jnp.float32),
            pltpu.SemaphoreType.DMA((_NBUF,)),
            pltpu.SemaphoreType.DMA((_NBUF,)),
        ],
        compiler_params=pltpu.CompilerParams(
            use_tc_tiling_on_sc=False, needs_layout_passes=False
        ),
    )
    def gather_kernel(weight_hbm, idx_hbm, out_hbm, idx_v, rows_v, gsem, osem):
        wid = lax.axis_index("s") * _NC + lax.axis_index("c")
        lane = lax.iota(jnp.int32, 16)
        zero16 = jnp.zeros((16,), jnp.float32)

        pltpu.sync_copy(idx_hbm.at[pl.ds(wid * idx_rows, idx_rows)], idx_v)

        def fire_gathers(t, b):
            for j in range(_CHUNK_STREAMS):
                pltpu.async_copy(
                    weight_hbm.at[idx_v.at[t * _CHUNK_STREAMS + j]],
                    rows_v.at[b].at[pl.ds(j * _STREAM, _STREAM)],
                    gsem.at[b],
                )

        def wait_gathers(b):
            pltpu.make_async_copy(
                weight_hbm.at[pl.ds(0, _CHUNK)], rows_v.at[b], gsem.at[b]
            ).wait()

        def fire_out(t, b):
            pltpu.async_copy(
                rows_v.at[b],
                out_hbm.at[pl.ds(wid * per_w + t * _CHUNK, _CHUNK)],
                osem.at[b],
            )

        def wait_out(b):
            pltpu.make_async_copy(
                rows_v.at[b], out_hbm.at[pl.ds(0, _CHUNK)], osem.at[b]
            ).wait()

        def fix_padding(t, b):
            row0 = t * _CHUNK_STREAMS
            m_any = idx_v[row0, pl.ds(0, 16)] == _PAD
            for g in range(1, _CHUNK // 16):
                r, o = divmod(g * 16, _STREAM)
                m_any = m_any | (idx_v[row0 + r, pl.ds(o, 16)] == _PAD)
            n_pad = jnp.sum(m_any.astype(jnp.int32))

            @pl.when(n_pad > 0)
            def _fix():
                for r in range(_CHUNK_STREAMS):
                    def fix_group(g, c):
                        idx16 = idx_v[row0 + r, pl.ds(g * 16, 16)]
                        m = idx16 == _PAD
                        @pl.when(jnp.sum(m.astype(jnp.int32)) > 0)
                        def _zero_rows():
                            rows16 = r * _STREAM + g * 16 + lane
                            for col in range(_D):
                                plsc.store_scatter(
                                    rows_v.at[b],
                                    [rows16, jnp.full((16,), col, jnp.int32)],
                                    zero16,
                                    mask=m,
                                )
                        return c
                    lax.fori_loop(0, _STREAM // 16, fix_group, 0)

        for t in range(_PF):
            fire_gathers(t, t % _NBUF)

        def ring_body(g, carry):
            for b in range(_NBUF):
                t = g * _NBUF + b
                wait_gathers(b)
                fix_padding(t, b)
                fire_out(t, b)
                bu = (b + _PF) % _NBUF
                t_pf = t + _PF

                @pl.when(t_pf >= _NBUF)
                def _drain():
                    wait_out(bu)

                @pl.when(t_pf < n_chunks)
                def _prefetch():
                    fire_gathers(t_pf, bu)
            return carry

        lax.fori_loop(0, n_chunks // _NBUF, ring_body, 0)

        for i in range(_PF):
            wait_out((n_chunks - _PF + i) % _NBUF)

    return gather_kernel


def kernel(indices, weight):
    B = indices.size
    V, D = weight.shape
    assert D == _D
    idx2d = indices.reshape(B // _STREAM, _STREAM).astype(jnp.int32)
    out = _make_gather(B, V)(weight, idx2d)
    return out.reshape(indices.shape + (D,))
